# baseline (speedup 1.0000x reference)
import functools

import jax
import jax.numpy as jnp
from jax import lax
from jax.experimental import pallas as pl
from jax.experimental.pallas import tpu as pltpu
from jax.experimental.pallas import tpu_sc as plsc

B = 4096
L = 50
D = 64
DW = D // 2
V = 21000
NUM_LT = 6
NUM_GT = 5
NUM_BAG = NUM_LT + NUM_GT
NUM_TAB = NUM_BAG + 1
NC, NS = 2, 16
NW = NC * NS
BW = B // NW
CH = 16
NCH = BW // CH
ROWS = CH * L


def _sc_embed(lt_tab, gt_tab, ids_flat, show_tab, show_ids):
    mesh = plsc.VectorSubcoreMesh(
        core_axis_name="c", subcore_axis_name="s", num_cores=NC, num_subcores=NS
    )

    @functools.partial(
        pl.kernel,
        out_type=jax.ShapeDtypeStruct((B, NUM_TAB * DW), jnp.uint32),
        mesh=mesh,
        scratch_types=[
            pltpu.VMEM((BW * L,), jnp.int32),
            pltpu.VMEM((ROWS, DW), jnp.uint32),
            pltpu.VMEM((ROWS, DW), jnp.uint32),
            pltpu.VMEM((BW, DW), jnp.uint32),
            pltpu.SemaphoreType.DMA,
            pltpu.SemaphoreType.DMA,
        ],
        compiler_params=pltpu.CompilerParams(use_tc_tiling_on_sc=False,
                                             needs_layout_passes=False),
    )
    def k(*refs):
        lt_t, gt_t = refs[0], refs[1]
        i_refs = refs[2:2 + NUM_BAG]
        show_t, show_i, out, idx_all, buf_a, buf_b, acc_v, sem_a, sem_b = (
            refs[2 + NUM_BAG:])

        wid = lax.axis_index("s") * NC + lax.axis_index("c")
        base = wid * BW

        def gather_start(tab, c, buf, sem):
            pltpu.async_copy(tab.at[idx_all.at[pl.ds(c * ROWS, ROWS)]],
                             buf, sem)

        def gather_wait(tab, c, buf, sem):
            pltpu.make_async_copy(tab.at[idx_all.at[pl.ds(c * ROWS, ROWS)]],
                                  buf, sem).wait()

        def compute_chunk(c, buf):
            def load(r, q):
                return plsc.bitcast(buf[r, pl.ds(16 * q, 16)], jnp.bfloat16)

            def bag_body(j, _):
                row0 = j * L
                accs = tuple(load(row0, q) for q in range(2))

                def l_body(i, a):
                    r = row0 + 1 + 2 * i
                    a = tuple(jnp.maximum(a[q], load(r, q)) for q in range(2))
                    return tuple(jnp.maximum(a[q], load(r + 1, q))
                                 for q in range(2))

                accs = lax.fori_loop(0, (L - 2) // 2, l_body, accs)
                accs = tuple(jnp.maximum(accs[q], load(row0 + L - 1, q))
                             for q in range(2))
                for q in range(2):
                    acc_v[c * CH + j, pl.ds(16 * q, 16)] = plsc.bitcast(
                        accs[q], jnp.uint32)
                return 0

            lax.fori_loop(0, CH, bag_body, 0)

        for t in range(NUM_BAG):
            tab = lt_t if t < NUM_LT else gt_t
            pltpu.sync_copy(i_refs[t].at[pl.ds(base * L, BW * L)], idx_all)
            gather_start(tab, 0, buf_a, sem_a)
            gather_start(tab, 1, buf_b, sem_b)

            def pipe(i, _, tab=tab):
                for p, (buf, sem) in enumerate(((buf_a, sem_a), (buf_b, sem_b))):
                    c = 2 * i + p
                    gather_wait(tab, c, buf, sem)
                    compute_chunk(c, buf)

                    @pl.when(c + 2 < NCH)
                    def _(c=c, buf=buf, sem=sem, tab=tab):
                        gather_start(tab, c + 2, buf, sem)
                return 0

            lax.fori_loop(0, NCH // 2, pipe, 0)
            pltpu.sync_copy(acc_v, out.at[pl.ds(base, BW), pl.ds(t * DW, DW)])

        pltpu.sync_copy(show_i.at[pl.ds(base, BW)], idx_all.at[pl.ds(0, BW)])
        pltpu.async_copy(show_t.at[idx_all.at[pl.ds(0, BW)]], acc_v,
                         sem_a).wait()
        pltpu.sync_copy(acc_v,
                        out.at[pl.ds(base, BW), pl.ds(NUM_BAG * DW, DW)])

    return k(lt_tab, gt_tab, *ids_flat, show_tab, show_ids)


def _pack_table(tab_f32, rows):
    b = tab_f32.astype(jnp.bfloat16).reshape(rows, DW, 2)
    return jax.lax.bitcast_convert_type(b, jnp.uint32)


def _tc_heads(emb_u32, w_even, w_odd, lin_b):
    nh = w_even.shape[0]

    def mm(emb_ref, we_ref, wo_ref, b_ref, out_ref):
        x = emb_ref[...]
        x_even = pltpu.bitcast(x << 16, jnp.float32)
        x_odd = pltpu.bitcast(x & jnp.uint32(0xFFFF0000), jnp.float32)
        for i in range(nh):
            out_ref[i] = (
                jnp.dot(x_even, we_ref[i], preferred_element_type=jnp.float32)
                + jnp.dot(x_odd, wo_ref[i], preferred_element_type=jnp.float32)
                + b_ref[i][None, :]
            )

    return pl.pallas_call(
        mm,
        out_shape=jax.ShapeDtypeStruct((nh, B, 5), jnp.float32),
    )(emb_u32, w_even, w_odd, lin_b)


def kernel(lt_ids_0, lt_ids_1, lt_ids_2, lt_ids_3, lt_ids_4, lt_ids_5,
           gt_ids_0, gt_ids_1, gt_ids_2, gt_ids_3, gt_ids_4,
           show_ids, lt_tables, gt_tables, show_table, lin_W, lin_b):
    lt_ids = [lt_ids_0, lt_ids_1, lt_ids_2, lt_ids_3, lt_ids_4, lt_ids_5]
    gt_ids = [gt_ids_0, gt_ids_1, gt_ids_2, gt_ids_3, gt_ids_4]
    ids_flat = [x.reshape(-1) + jnp.int32(t * V)
                for t, x in enumerate(lt_ids)]
    ids_flat += [x.reshape(-1) + jnp.int32(t * V)
                 for t, x in enumerate(gt_ids)]
    emb_u32 = _sc_embed(
        _pack_table(lt_tables, NUM_LT * V),
        _pack_table(gt_tables, NUM_GT * V),
        ids_flat,
        _pack_table(show_table, V),
        show_ids,
    )
    return _tc_heads(emb_u32, lin_W[:, 0::2, :], lin_W[:, 1::2, :], lin_b)


# speedup vs baseline: 1.3041x; 1.3041x over previous
import functools

import jax
import jax.numpy as jnp
from jax import lax
from jax.experimental import pallas as pl
from jax.experimental.pallas import tpu as pltpu
from jax.experimental.pallas import tpu_sc as plsc

B = 4096
L = 50
D = 64
V = 21000
NUM_LT = 6
NUM_GT = 5
NC, NS = 2, 16
NW = NC * NS
BW = B // NW
CH = 16
NCH = BW // CH
ROWS = CH * L
NT_G = 4


def _sc_group(tab, ids_flat, show_tab, show_ids):
    nbag = NT_G if show_tab is None else NT_G - 1
    extra = () if show_tab is None else (show_tab, show_ids)
    mesh = plsc.VectorSubcoreMesh(
        core_axis_name="c", subcore_axis_name="s", num_cores=NC, num_subcores=NS
    )

    @functools.partial(
        pl.kernel,
        out_type=jax.ShapeDtypeStruct((B, NT_G * D), jnp.float32),
        mesh=mesh,
        scratch_types=[
            pltpu.VMEM((BW * L,), jnp.int32),
            pltpu.VMEM((ROWS, D), jnp.float32),
            pltpu.VMEM((ROWS, D), jnp.float32),
            pltpu.VMEM((BW, D), jnp.float32),
            pltpu.SemaphoreType.DMA,
            pltpu.SemaphoreType.DMA,
        ],
        compiler_params=pltpu.CompilerParams(use_tc_tiling_on_sc=False),
    )
    def k(*refs):
        if show_tab is None:
            tab_r, ids_r, out = refs[0], refs[1], refs[2]
            idx_all, buf_a, buf_b, acc_v, sem_a, sem_b = refs[3:]
            show_t = show_i = None
        else:
            tab_r, ids_r, show_t, show_i, out = refs[:5]
            idx_all, buf_a, buf_b, acc_v, sem_a, sem_b = refs[5:]

        wid = lax.axis_index("s") * NC + lax.axis_index("c")
        base = wid * BW

        def gather_start(c, buf, sem):
            pltpu.async_copy(tab_r.at[idx_all.at[pl.ds(c * ROWS, ROWS)]],
                             buf, sem)

        def gather_wait(c, buf, sem):
            pltpu.make_async_copy(tab_r.at[idx_all.at[pl.ds(c * ROWS, ROWS)]],
                                  buf, sem).wait()

        def compute_chunk(c, buf):
            def bag_body(j, _):
                row0 = j * L
                accs = tuple(buf[row0, pl.ds(16 * q, 16)] for q in range(4))

                def l_body(i, a):
                    r = row0 + 1 + 2 * i
                    a = tuple(jnp.maximum(a[q], buf[r, pl.ds(16 * q, 16)])
                              for q in range(4))
                    return tuple(jnp.maximum(a[q], buf[r + 1, pl.ds(16 * q, 16)])
                                 for q in range(4))

                accs = lax.fori_loop(0, (L - 2) // 2, l_body, accs)
                accs = tuple(jnp.maximum(accs[q],
                                         buf[row0 + L - 1, pl.ds(16 * q, 16)])
                             for q in range(4))
                for q in range(4):
                    acc_v[c * CH + j, pl.ds(16 * q, 16)] = accs[q]
                return 0

            lax.fori_loop(0, CH, bag_body, 0)

        for t in range(nbag):
            pltpu.sync_copy(ids_r.at[pl.ds((t * B + base) * L, BW * L)],
                            idx_all)
            gather_start(0, buf_a, sem_a)
            gather_start(1, buf_b, sem_b)

            def pipe(i, _):
                for p, (buf, sem) in enumerate(((buf_a, sem_a), (buf_b, sem_b))):
                    c = 2 * i + p
                    gather_wait(c, buf, sem)
                    compute_chunk(c, buf)

                    @pl.when(c + 2 < NCH)
                    def _(c=c, buf=buf, sem=sem):
                        gather_start(c + 2, buf, sem)
                return 0

            lax.fori_loop(0, NCH // 2, pipe, 0)
            pltpu.sync_copy(acc_v, out.at[pl.ds(base, BW), pl.ds(t * D, D)])

        if show_tab is not None:
            pltpu.sync_copy(show_i.at[pl.ds(base, BW)],
                            idx_all.at[pl.ds(0, BW)])
            pltpu.async_copy(show_t.at[idx_all.at[pl.ds(0, BW)]], acc_v,
                             sem_a).wait()
            pltpu.sync_copy(acc_v,
                            out.at[pl.ds(base, BW), pl.ds(nbag * D, D)])

    return k(tab, ids_flat, *extra)


def _tc_heads(embs, w_parts, lin_b):
    nh = lin_b.shape[0]

    def mm(e0_ref, e1_ref, e2_ref, w0_ref, w1_ref, w2_ref, b_ref, out_ref):
        es = (e0_ref[...], e1_ref[...], e2_ref[...])
        ws = (w0_ref, w1_ref, w2_ref)
        for i in range(nh):
            acc = b_ref[i][None, :]
            for g in range(3):
                acc = acc + jnp.dot(es[g], ws[g][i],
                                    preferred_element_type=jnp.float32)
            out_ref[i] = acc

    return pl.pallas_call(
        mm,
        out_shape=jax.ShapeDtypeStruct((nh, B, 5), jnp.float32),
    )(*embs, *w_parts, lin_b)


def kernel(lt_ids_0, lt_ids_1, lt_ids_2, lt_ids_3, lt_ids_4, lt_ids_5,
           gt_ids_0, gt_ids_1, gt_ids_2, gt_ids_3, gt_ids_4,
           show_ids, lt_tables, gt_tables, show_table, lin_W, lin_b):
    all_ids = [lt_ids_0, lt_ids_1, lt_ids_2, lt_ids_3, lt_ids_4, lt_ids_5,
               gt_ids_0, gt_ids_1, gt_ids_2, gt_ids_3, gt_ids_4]
    all_tabs = [lt_tables[i] for i in range(NUM_LT)]
    all_tabs += [gt_tables[i] for i in range(NUM_GT)]

    def group_inputs(ts):
        tab = jnp.concatenate([all_tabs[t] for t in ts], axis=0)
        ids = jnp.concatenate(
            [all_ids[t].reshape(-1) + jnp.int32(k * V)
             for k, t in enumerate(ts)])
        return tab, ids

    groups = [(0, 1, 2, 3), (4, 5, 6, 7), (8, 9, 10)]
    embs = []
    for gi, ts in enumerate(groups):
        tab, ids = group_inputs(ts)
        if gi < 2:
            embs.append(_sc_group(tab, ids, None, None))
        else:
            embs.append(_sc_group(tab, ids, show_table, show_ids))

    w_parts = [lin_W[:, g * NT_G * D:(g + 1) * NT_G * D, :] for g in range(3)]
    return _tc_heads(embs, w_parts, lin_b)


# speedup vs baseline: 1.6750x; 1.2844x over previous
import functools

import jax
import jax.numpy as jnp
from jax import lax
from jax.experimental import pallas as pl
from jax.experimental.pallas import tpu as pltpu
from jax.experimental.pallas import tpu_sc as plsc

B = 4096
L = 50
D = 64
V = 21000
NUM_LT = 6
NUM_GT = 5
NUM_BAG = NUM_LT + NUM_GT
NUM_TAB = NUM_BAG + 1
NC, NS = 2, 16
NW = NC * NS
BW = B // NW
CH = 16
NCH = BW // CH
ROWS = CH * L


def _sc_group(tab_s, ids_flat, nbag, show_tab=None, show_ids=None):
    ntab = nbag + (0 if show_tab is None else 1)
    extra = () if show_tab is None else (show_tab, show_ids)
    mesh = plsc.VectorSubcoreMesh(
        core_axis_name="c", subcore_axis_name="s", num_cores=NC, num_subcores=NS
    )

    @functools.partial(
        pl.kernel,
        out_type=jax.ShapeDtypeStruct((B, ntab * D), jnp.bfloat16),
        mesh=mesh,
        scratch_types=[
            pltpu.VMEM((BW * L,), jnp.int32),
            pltpu.VMEM((ROWS, D), jnp.bfloat16),
            pltpu.VMEM((ROWS, D), jnp.bfloat16),
            pltpu.VMEM((BW, D), jnp.bfloat16),
            pltpu.SemaphoreType.DMA,
            pltpu.SemaphoreType.DMA,
        ],
        compiler_params=pltpu.CompilerParams(use_tc_tiling_on_sc=False),
    )
    def k(*refs):
        if show_tab is None:
            tab, ids, out = refs[:3]
            idx_all, buf_a, buf_b, acc_v, sem_a, sem_b = refs[3:]
            show_t = show_i = None
        else:
            tab, ids, show_t, show_i, out = refs[:5]
            idx_all, buf_a, buf_b, acc_v, sem_a, sem_b = refs[5:]
        wid = lax.axis_index("s") * NC + lax.axis_index("c")
        base = wid * BW

        def gather_start(tab, c, buf, sem):
            pltpu.async_copy(tab.at[idx_all.at[pl.ds(c * ROWS, ROWS)]],
                             buf, sem)

        def gather_wait(tab, c, buf, sem):
            pltpu.make_async_copy(tab.at[idx_all.at[pl.ds(c * ROWS, ROWS)]],
                                  buf, sem).wait()

        def compute_chunk(c, buf):
            def bag_body(j, _):
                row0 = j * L
                accs = tuple(buf[row0, pl.ds(32 * q, 32)] for q in range(2))

                def l_body(i, a):
                    r = row0 + 1 + 2 * i
                    a = tuple(jnp.maximum(a[q], buf[r, pl.ds(32 * q, 32)])
                              for q in range(2))
                    return tuple(jnp.maximum(a[q], buf[r + 1, pl.ds(32 * q, 32)])
                                 for q in range(2))

                accs = lax.fori_loop(0, (L - 2) // 2, l_body, accs)
                accs = tuple(jnp.maximum(accs[q],
                                         buf[row0 + L - 1, pl.ds(32 * q, 32)])
                             for q in range(2))
                for q in range(2):
                    acc_v[c * CH + j, pl.ds(32 * q, 32)] = accs[q]
                return 0

            lax.fori_loop(0, CH, bag_body, 0)

        for t in range(nbag):
            pltpu.sync_copy(ids.at[pl.ds(t * B * L + base * L, BW * L)],
                            idx_all)
            gather_start(tab, 0, buf_a, sem_a)
            gather_start(tab, 1, buf_b, sem_b)

            def pipe(i, _, tab=tab):
                for p, (buf, sem) in enumerate(((buf_a, sem_a), (buf_b, sem_b))):
                    c = 2 * i + p
                    gather_wait(tab, c, buf, sem)
                    compute_chunk(c, buf)

                    @pl.when(c + 2 < NCH)
                    def _(c=c, buf=buf, sem=sem, tab=tab):
                        gather_start(tab, c + 2, buf, sem)
                return 0

            lax.fori_loop(0, NCH // 2, pipe, 0)
            pltpu.sync_copy(acc_v, out.at[pl.ds(base, BW), pl.ds(t * D, D)])

        if show_tab is not None:
            pltpu.sync_copy(show_i.at[pl.ds(base, BW)],
                            idx_all.at[pl.ds(0, BW)])
            pltpu.async_copy(show_t.at[idx_all.at[pl.ds(0, BW)]], acc_v,
                             sem_a).wait()
            pltpu.sync_copy(acc_v,
                            out.at[pl.ds(base, BW), pl.ds(nbag * D, D)])

    return k(tab_s, ids_flat, *extra)


def _tc_heads(emb_lt, emb_gt, lin_W, lin_b):
    def mm(e0_ref, e1_ref, w_ref, b_ref, out_ref):
        x = jnp.concatenate([e0_ref[...], e1_ref[...]],
                            axis=1).astype(jnp.float32)
        for i in range(lin_W.shape[0]):
            out_ref[i] = (
                jnp.dot(x, w_ref[i], preferred_element_type=jnp.float32)
                + b_ref[i][None, :]
            )

    return pl.pallas_call(
        mm,
        out_shape=jax.ShapeDtypeStruct((lin_W.shape[0], B, 5), jnp.float32),
    )(emb_lt, emb_gt, lin_W, lin_b)


def kernel(lt_ids_0, lt_ids_1, lt_ids_2, lt_ids_3, lt_ids_4, lt_ids_5,
           gt_ids_0, gt_ids_1, gt_ids_2, gt_ids_3, gt_ids_4,
           show_ids, lt_tables, gt_tables, show_table, lin_W, lin_b):
    lt_ids = jnp.stack([lt_ids_0, lt_ids_1, lt_ids_2, lt_ids_3, lt_ids_4,
                        lt_ids_5]).reshape(NUM_LT, B * L)
    gt_ids = jnp.stack([gt_ids_0, gt_ids_1, gt_ids_2, gt_ids_3,
                        gt_ids_4]).reshape(NUM_GT, B * L)
    lt_off = (jnp.arange(NUM_LT, dtype=jnp.int32) * V)[:, None]
    gt_off = (jnp.arange(NUM_GT, dtype=jnp.int32) * V)[:, None]
    emb_lt = _sc_group(
        lt_tables.astype(jnp.bfloat16).reshape(NUM_LT * V, D),
        (lt_ids + lt_off).reshape(-1), NUM_LT)
    emb_gt = _sc_group(
        gt_tables.astype(jnp.bfloat16).reshape(NUM_GT * V, D),
        (gt_ids + gt_off).reshape(-1), NUM_GT,
        show_table.astype(jnp.bfloat16), show_ids)
    return _tc_heads(emb_lt, emb_gt, lin_W, lin_b)


# speedup vs baseline: 1.7468x; 1.0429x over previous
import functools

import jax
import jax.numpy as jnp
from jax import lax
from jax.experimental import pallas as pl
from jax.experimental.pallas import tpu as pltpu
from jax.experimental.pallas import tpu_sc as plsc

B = 4096
L = 50
D = 64
V = 21000
NUM_LT = 6
NUM_GT = 5
NUM_BAG = NUM_LT + NUM_GT
NUM_TAB = NUM_BAG + 1
NC, NS = 2, 16
NW = NC * NS
BW = B // NW
CH = 16
NCH = BW // CH
ROWS = CH * L


def _sc_group(tab_s, ids_flat, nbag, show_tab=None, show_ids=None):
    ntab = nbag + (0 if show_tab is None else 1)
    extra = () if show_tab is None else (show_tab, show_ids)
    mesh = plsc.VectorSubcoreMesh(
        core_axis_name="c", subcore_axis_name="s", num_cores=NC, num_subcores=NS
    )

    @functools.partial(
        pl.kernel,
        out_type=jax.ShapeDtypeStruct((B, ntab * D), jnp.bfloat16),
        mesh=mesh,
        scratch_types=[
            pltpu.VMEM((BW * L,), jnp.int32),
            pltpu.VMEM((ROWS, D), jnp.bfloat16),
            pltpu.VMEM((ROWS, D), jnp.bfloat16),
            pltpu.VMEM((BW, D), jnp.bfloat16),
            pltpu.SemaphoreType.DMA,
            pltpu.SemaphoreType.DMA,
        ],
        compiler_params=pltpu.CompilerParams(use_tc_tiling_on_sc=False),
    )
    def k(*refs):
        if show_tab is None:
            tab, ids, out = refs[:3]
            idx_all, buf_a, buf_b, acc_v, sem_a, sem_b = refs[3:]
            show_t = show_i = None
        else:
            tab, ids, show_t, show_i, out = refs[:5]
            idx_all, buf_a, buf_b, acc_v, sem_a, sem_b = refs[5:]
        wid = lax.axis_index("s") * NC + lax.axis_index("c")
        base = wid * BW

        def gather_start(tab, c, buf, sem):
            pltpu.async_copy(tab.at[idx_all.at[pl.ds(c * ROWS, ROWS)]],
                             buf, sem)

        def gather_wait(tab, c, buf, sem):
            pltpu.make_async_copy(tab.at[idx_all.at[pl.ds(c * ROWS, ROWS)]],
                                  buf, sem).wait()

        def compute_chunk(c, buf):
            def bag_body(j, _):
                row0 = j * L
                accs = tuple(buf[row0, pl.ds(32 * q, 32)] for q in range(2))

                def l_body(i, a):
                    r = row0 + 1 + 2 * i
                    a = tuple(jnp.maximum(a[q], buf[r, pl.ds(32 * q, 32)])
                              for q in range(2))
                    return tuple(jnp.maximum(a[q], buf[r + 1, pl.ds(32 * q, 32)])
                                 for q in range(2))

                accs = lax.fori_loop(0, (L - 2) // 2, l_body, accs)
                accs = tuple(jnp.maximum(accs[q],
                                         buf[row0 + L - 1, pl.ds(32 * q, 32)])
                             for q in range(2))
                for q in range(2):
                    acc_v[c * CH + j, pl.ds(32 * q, 32)] = accs[q]
                return 0

            lax.fori_loop(0, CH, bag_body, 0)

        for t in range(nbag):
            pltpu.sync_copy(ids.at[pl.ds(t * B * L + base * L, BW * L)],
                            idx_all)
            gather_start(tab, 0, buf_a, sem_a)
            gather_start(tab, 1, buf_b, sem_b)

            def pipe(i, _, tab=tab):
                for p, (buf, sem) in enumerate(((buf_a, sem_a), (buf_b, sem_b))):
                    c = 2 * i + p
                    gather_wait(tab, c, buf, sem)
                    compute_chunk(c, buf)

                    @pl.when(c + 2 < NCH)
                    def _(c=c, buf=buf, sem=sem, tab=tab):
                        gather_start(tab, c + 2, buf, sem)
                return 0

            lax.fori_loop(0, NCH // 2, pipe, 0)
            pltpu.sync_copy(acc_v, out.at[pl.ds(base, BW), pl.ds(t * D, D)])

        if show_tab is not None:
            pltpu.sync_copy(show_i.at[pl.ds(base, BW)],
                            idx_all.at[pl.ds(0, BW)])
            pltpu.async_copy(show_t.at[idx_all.at[pl.ds(0, BW)]], acc_v,
                             sem_a).wait()
            pltpu.sync_copy(acc_v,
                            out.at[pl.ds(base, BW), pl.ds(nbag * D, D)])

    return k(tab_s, ids_flat, *extra)


def _tc_heads(embs, lin_W, lin_b):
    def mm(e0_ref, e1_ref, e2_ref, e3_ref, w_ref, b_ref, out_ref):
        x = jnp.concatenate(
            [e0_ref[...], e1_ref[...], e2_ref[...], e3_ref[...]],
            axis=1).astype(jnp.float32)
        for i in range(lin_W.shape[0]):
            out_ref[i] = (
                jnp.dot(x, w_ref[i], preferred_element_type=jnp.float32)
                + b_ref[i][None, :]
            )

    return pl.pallas_call(
        mm,
        out_shape=jax.ShapeDtypeStruct((lin_W.shape[0], B, 5), jnp.float32),
    )(*embs, lin_W, lin_b)


def kernel(lt_ids_0, lt_ids_1, lt_ids_2, lt_ids_3, lt_ids_4, lt_ids_5,
           gt_ids_0, gt_ids_1, gt_ids_2, gt_ids_3, gt_ids_4,
           show_ids, lt_tables, gt_tables, show_table, lin_W, lin_b):
    lt_ids = [lt_ids_0, lt_ids_1, lt_ids_2, lt_ids_3, lt_ids_4, lt_ids_5]
    gt_ids = [gt_ids_0, gt_ids_1, gt_ids_2, gt_ids_3, gt_ids_4]
    off3 = (jnp.arange(3, dtype=jnp.int32) * V)[:, None]
    off2 = (jnp.arange(2, dtype=jnp.int32) * V)[:, None]

    def stack_ids(idl, off):
        ids = jnp.stack(idl).reshape(len(idl), B * L)
        return (ids + off).reshape(-1)

    embs = [
        _sc_group(
            lt_tables[0:3].astype(jnp.bfloat16).reshape(3 * V, D),
            stack_ids(lt_ids[0:3], off3), 3),
        _sc_group(
            lt_tables[3:6].astype(jnp.bfloat16).reshape(3 * V, D),
            stack_ids(lt_ids[3:6], off3), 3),
        _sc_group(
            gt_tables[0:3].astype(jnp.bfloat16).reshape(3 * V, D),
            stack_ids(gt_ids[0:3], off3), 3),
        _sc_group(
            gt_tables[3:5].astype(jnp.bfloat16).reshape(2 * V, D),
            stack_ids(gt_ids[3:5], off2), 2,
            show_table.astype(jnp.bfloat16), show_ids),
    ]
    return _tc_heads(embs, lin_W, lin_b)
